# Initial kernel scaffold; baseline (speedup 1.0000x reference)
import jax
import jax.numpy as jnp
from jax.experimental import pallas as pl


def kernel(t, z, edge_index, Wl, Wr, att, bias):
    raise NotImplementedError("write your pallas kernel here")


# speedup vs baseline: 2.2799x; 2.2799x over previous
import jax
import jax.numpy as jnp
from jax import lax
from jax.experimental import pallas as pl

_N = 307


def _gat_body(x_ref, src_ref, dst_ref, wl_ref, wr_ref, att_ref, bias_ref, o_ref):
    xi = x_ref[0]
    xl = jnp.dot(xi, wl_ref[...], preferred_element_type=jnp.float32)
    xr = jnp.dot(xi, wr_ref[...], preferred_element_type=jnp.float32)
    srcc = src_ref[...]
    dstc = dst_ref[...]
    et = srcc.shape[0]
    n = xi.shape[0]
    iota = lax.broadcasted_iota(jnp.int32, (et, n), 1)
    S = (srcc == iota).astype(jnp.float32)
    D = (dstc == iota).astype(jnp.float32)
    xls = jnp.dot(S, xl, preferred_element_type=jnp.float32)
    xrd = jnp.dot(D, xr, preferred_element_type=jnp.float32)
    m = xls + xrd
    h = jnp.where(m >= 0, m, 0.2 * m)
    e = jnp.dot(h, att_ref[...], preferred_element_type=jnp.float32)
    masked = jnp.where(D > 0, e, -1e30)
    emax = jnp.max(masked, axis=0)
    emax_e = jnp.sum(D * emax[None, :], axis=1, keepdims=True)
    ex = jnp.exp(e - emax_e)
    den = jnp.sum(D * ex, axis=0)
    den_e = jnp.sum(D * den[None, :], axis=1, keepdims=True)
    alpha = ex / (den_e + 1e-16)
    w = alpha * xls
    out = lax.dot_general(D, w, (((0,), (0,)), ((), ())),
                          preferred_element_type=jnp.float32)
    o_ref[0] = out + bias_ref[...]


def kernel(t, z, edge_index, Wl, Wr, att, bias):
    n = _N
    h = z.shape[1]
    b = z.shape[0] // n
    e = edge_index.shape[1]
    et = e + n
    loop = jnp.arange(n, dtype=edge_index.dtype)
    src = jnp.concatenate([edge_index[0], loop]).reshape(et, 1)
    dst = jnp.concatenate([edge_index[1], loop]).reshape(et, 1)
    x = z.reshape(b, n, h)
    out = pl.pallas_call(
        _gat_body,
        grid=(b,),
        in_specs=[
            pl.BlockSpec((1, n, h), lambda i: (i, 0, 0)),
            pl.BlockSpec((et, 1), lambda i: (0, 0)),
            pl.BlockSpec((et, 1), lambda i: (0, 0)),
            pl.BlockSpec((h, h), lambda i: (0, 0)),
            pl.BlockSpec((h, h), lambda i: (0, 0)),
            pl.BlockSpec((h, 1), lambda i: (0, 0)),
            pl.BlockSpec((1, h), lambda i: (0, 0)),
        ],
        out_specs=pl.BlockSpec((1, n, h), lambda i: (i, 0, 0)),
        out_shape=jax.ShapeDtypeStruct((b, n, h), jnp.float32),
    )(x, src, dst, Wl.T, Wr.T, att.reshape(h, 1), bias.reshape(1, h))
    return out.reshape(b * n, h, 1)
